# Initial kernel scaffold; baseline (speedup 1.0000x reference)
import jax
import jax.numpy as jnp
from jax.experimental import pallas as pl


def kernel(edge_index, edge_edge_index, motif_edge_index, node_embeds, W_h, b_h, W_l1, b_l1, W_gcn, b_gcn, W_l2, b_l2, W_eo, b_eo, W_l3, b_l3, W_o, b_o):
    raise NotImplementedError("write your pallas kernel here")


# speedup vs baseline: 32.2022x; 32.2022x over previous
import functools

import jax
import jax.numpy as jnp
from jax.experimental import pallas as pl
from jax.experimental.pallas import tpu as pltpu

_NB = 256
_CHUNK = 2048


def _hist_kernel(s1, d1, s2, d2, s3, d3, c1, c2, c3):
    step = pl.program_id(0)

    def onehot(idx_ref):
        idx = idx_ref[0]
        rows = jax.lax.broadcasted_iota(jnp.int32, (_NB, _CHUNK), 0)
        return jnp.where(idx == rows, 1.0, 0.0).astype(jnp.bfloat16)

    dn = (((1,), (1,)), ((), ()))
    for sref, dref, cref in ((s1, d1, c1), (s2, d2, c2), (s3, d3, c3)):
        oh_s = onehot(sref)
        oh_d = onehot(dref)
        prod = jax.lax.dot_general(oh_d, oh_s, dn,
                                   preferred_element_type=jnp.float32)

        @pl.when(step == 0)
        def _(cref=cref, prod=prod):
            cref[...] = prod

        @pl.when(step != 0)
        def _(cref=cref, prod=prod):
            cref[...] += prod


def _histograms(src1, dst1, src2, dst2, src3, dst3):
    e = src1.shape[0]
    steps = e // _CHUNK
    r = lambda a: a.reshape(steps, 1, _CHUNK)
    in_spec = pl.BlockSpec((1, 1, _CHUNK), lambda i: (i, 0, 0))
    out_spec = pl.BlockSpec((_NB, _NB), lambda i: (0, 0))
    return pl.pallas_call(
        _hist_kernel,
        grid=(steps,),
        in_specs=[in_spec] * 6,
        out_specs=[out_spec] * 3,
        out_shape=[jax.ShapeDtypeStruct((_NB, _NB), jnp.float32)] * 3,
    )(r(src1), r(dst1), r(src2), r(dst2), r(src3), r(dst3))


def _leaky(v):
    return jnp.where(v >= 0, v, 0.01 * v)


def _dense_kernel(c1, c2, c3, ne, w_h, b_h, w_l1, b_l1, w_gcn, b_gcn,
                  w_l2, b_l2, w_l3, b_l3, w_o, b_o, out, yt_scr):
    f32 = jnp.float32
    dn_t = (((0,), (0,)), ((), ()))
    ones_col = jnp.ones((_NB, 1), f32)

    C = c1[...]
    A = c2[...]
    Mc = c3[...]

    Bdeg = jnp.dot(C, ones_col, preferred_element_type=f32)
    Ddeg = jax.lax.dot_general(C, ones_col, dn_t,
                               preferred_element_type=f32)
    Binv = jnp.where(Bdeg > 0, 1.0 / Bdeg, 0.0)
    Dinv = jnp.where(Ddeg > 0, 1.0 / Ddeg, 0.0)

    x = jnp.dot(ne[...], w_h[...], preferred_element_type=f32)
    e = jnp.dot(C, x, preferred_element_type=f32) * Binv
    y = jax.lax.dot_general(C, e, dn_t, preferred_element_type=f32) * Dinv + b_h[...]
    y = _leaky(y)
    y = jax.lax.dot_general(y, y, dn_t, preferred_element_type=f32)
    y = _leaky(jnp.dot(y, w_l1[...], preferred_element_type=f32) + b_l1[...])

    ymean = jnp.dot(C, y, preferred_element_type=f32) / jnp.maximum(Bdeg, 1.0)
    xw = jnp.dot(ymean, w_gcn[...], preferred_element_type=f32)
    deg = jnp.dot(A, ones_col, preferred_element_type=f32) + 1.0
    dis = jnp.where(deg > 0, jax.lax.rsqrt(deg), 0.0)
    z = dis * xw
    yg = dis * (jnp.dot(A, z, preferred_element_type=f32) + z) + b_gcn[...]
    y = _leaky(jnp.dot(yg, w_l2[...], preferred_element_type=f32) + b_l2[...])

    pen = jnp.where(Mc > 0, 0.0, jnp.float32(jnp.inf))
    i0 = jax.lax.broadcasted_iota(jnp.int32, (_NB, _NB), 0)
    i1 = jax.lax.broadcasted_iota(jnp.int32, (_NB, _NB), 1)
    eye = jnp.where(i0 == i1, 1.0, 0.0)
    yt_scr[...] = jax.lax.dot_general(y, eye, dn_t,
                                      preferred_element_type=f32)

    def jstep(j, acc):
        yrow = yt_scr[pl.ds(j, 1), :]
        aj = pen + yrow
        colmin = jnp.min(aj, axis=1, keepdims=True)
        colmin = jnp.where(jnp.isfinite(colmin), colmin, 0.0)
        wrow = w_l3[pl.ds(j, 1), :]
        return acc + jnp.dot(colmin, wrow, preferred_element_type=f32)

    ym = jax.lax.fori_loop(0, _NB, jstep,
                           jnp.zeros((_NB, 128), f32)) + b_l3[...]
    out[...] = jnp.dot(ym, w_o[...], preferred_element_type=f32) + b_o[...]


def _dense(counts, ne, *weights):
    c1, c2, c3 = counts
    args = (c1, c2, c3, ne) + tuple(weights)
    return pl.pallas_call(
        _dense_kernel,
        out_shape=jax.ShapeDtypeStruct((_NB, 64), jnp.float32),
        scratch_shapes=[pltpu.VMEM((_NB, _NB), jnp.float32)],
    )(*args)


def kernel(edge_index, edge_edge_index, motif_edge_index, node_embeds,
           W_h, b_h, W_l1, b_l1, W_gcn, b_gcn, W_l2, b_l2, W_eo, b_eo,
           W_l3, b_l3, W_o, b_o):
    del W_eo, b_eo
    counts = _histograms(edge_index[0], edge_index[1],
                         edge_edge_index[0], edge_edge_index[1],
                         motif_edge_index[0], motif_edge_index[1])
    row = lambda b: b.reshape(1, -1)
    return _dense(counts, node_embeds,
                  W_h, row(b_h), W_l1, row(b_l1), W_gcn, row(b_gcn),
                  W_l2, row(b_l2), W_l3, row(b_l3), W_o, row(b_o))


# speedup vs baseline: 40.2399x; 1.2496x over previous
import functools

import jax
import jax.numpy as jnp
from jax.experimental import pallas as pl
from jax.experimental.pallas import tpu as pltpu

_NB = 256
_CHUNK = 2048


def _hist_kernel(s1, d1, s2, d2, s3, d3, c1, c2, c3):
    step = pl.program_id(0)

    def onehot(idx_ref):
        idx = idx_ref[0]
        rows = jax.lax.broadcasted_iota(jnp.int32, (_NB, _CHUNK), 0)
        return jnp.where(idx == rows, 1.0, 0.0).astype(jnp.bfloat16)

    dn = (((1,), (1,)), ((), ()))
    for sref, dref, cref in ((s1, d1, c1), (s2, d2, c2), (s3, d3, c3)):
        oh_s = onehot(sref)
        oh_d = onehot(dref)
        prod = jax.lax.dot_general(oh_d, oh_s, dn,
                                   preferred_element_type=jnp.float32)

        @pl.when(step == 0)
        def _(cref=cref, prod=prod):
            cref[...] = prod

        @pl.when(step != 0)
        def _(cref=cref, prod=prod):
            cref[...] += prod


def _histograms(src1, dst1, src2, dst2, src3, dst3):
    e = src1.shape[0]
    steps = e // _CHUNK
    r = lambda a: a.reshape(steps, 1, _CHUNK)
    in_spec = pl.BlockSpec((1, 1, _CHUNK), lambda i: (i, 0, 0))
    out_spec = pl.BlockSpec((_NB, _NB), lambda i: (0, 0))
    return pl.pallas_call(
        _hist_kernel,
        grid=(steps,),
        in_specs=[in_spec] * 6,
        out_specs=[out_spec] * 3,
        out_shape=[jax.ShapeDtypeStruct((_NB, _NB), jnp.float32)] * 3,
    )(r(src1), r(dst1), r(src2), r(dst2), r(src3), r(dst3))


def _leaky(v):
    return jnp.where(v >= 0, v, 0.01 * v)


def _dense_kernel(c1, c2, c3, ne, w_h, b_h, w_l1, b_l1, w_gcn, b_gcn,
                  w_l2, b_l2, w_l3, b_l3, w_o, b_o, out, yt_scr):
    f32 = jnp.float32
    dn_t = (((0,), (0,)), ((), ()))
    ones_col = jnp.ones((_NB, 1), f32)

    C = c1[...]
    A = c2[...]
    Mc = c3[...]

    Bdeg = jnp.dot(C, ones_col, preferred_element_type=f32)
    Ddeg = jax.lax.dot_general(C, ones_col, dn_t,
                               preferred_element_type=f32)
    Binv = jnp.where(Bdeg > 0, 1.0 / Bdeg, 0.0)
    Dinv = jnp.where(Ddeg > 0, 1.0 / Ddeg, 0.0)

    x = jnp.dot(ne[...], w_h[...], preferred_element_type=f32)
    e = jnp.dot(C, x, preferred_element_type=f32) * Binv
    y = jax.lax.dot_general(C, e, dn_t, preferred_element_type=f32) * Dinv + b_h[...]
    y = _leaky(y)
    y = jax.lax.dot_general(y, y, dn_t, preferred_element_type=f32)
    y = _leaky(jnp.dot(y, w_l1[...], preferred_element_type=f32) + b_l1[...])

    ymean = jnp.dot(C, y, preferred_element_type=f32) / jnp.maximum(Bdeg, 1.0)
    xw = jnp.dot(ymean, w_gcn[...], preferred_element_type=f32)
    deg = jnp.dot(A, ones_col, preferred_element_type=f32) + 1.0
    dis = jnp.where(deg > 0, jax.lax.rsqrt(deg), 0.0)
    z = dis * xw
    yg = dis * (jnp.dot(A, z, preferred_element_type=f32) + z) + b_gcn[...]
    y = _leaky(jnp.dot(yg, w_l2[...], preferred_element_type=f32) + b_l2[...])

    big = jnp.float32(1e30)
    pen = jnp.where(Mc > 0, 0.0, big)
    yt_scr[...] = y

    def nstep(nb, acc):
        sel0 = jax.lax.broadcasted_iota(jnp.int32, (_NB, 8), 0)
        sel1 = jax.lax.broadcasted_iota(jnp.int32, (_NB, 8), 1)
        sel = jnp.where(sel0 == nb * 8 + sel1, 1.0, 0.0)
        pcols = jnp.dot(pen, sel, preferred_element_type=f32)
        yblk = yt_scr[pl.ds(nb * 8, 8), :]
        for u in range(8):
            cand = pcols[:, u:u + 1] + yblk[u:u + 1, :]
            acc = jnp.minimum(acc, cand)
        return acc

    acc = jax.lax.fori_loop(0, _NB // 8, nstep,
                            jnp.full((_NB, _NB), big, f32))
    ymin = jnp.where(acc > 1e29, 0.0, acc)
    ym = jnp.dot(ymin, w_l3[...], preferred_element_type=f32) + b_l3[...]
    out[...] = jnp.dot(ym, w_o[...], preferred_element_type=f32) + b_o[...]


def _dense(counts, ne, *weights):
    c1, c2, c3 = counts
    args = (c1, c2, c3, ne) + tuple(weights)
    return pl.pallas_call(
        _dense_kernel,
        out_shape=jax.ShapeDtypeStruct((_NB, 64), jnp.float32),
        scratch_shapes=[pltpu.VMEM((_NB, _NB), jnp.float32)],
    )(*args)


def kernel(edge_index, edge_edge_index, motif_edge_index, node_embeds,
           W_h, b_h, W_l1, b_l1, W_gcn, b_gcn, W_l2, b_l2, W_eo, b_eo,
           W_l3, b_l3, W_o, b_o):
    del W_eo, b_eo
    counts = _histograms(edge_index[0], edge_index[1],
                         edge_edge_index[0], edge_edge_index[1],
                         motif_edge_index[0], motif_edge_index[1])
    row = lambda b: b.reshape(1, -1)
    return _dense(counts, node_embeds,
                  W_h, row(b_h), W_l1, row(b_l1), W_gcn, row(b_gcn),
                  W_l2, row(b_l2), W_l3, row(b_l3), W_o, row(b_o))


# speedup vs baseline: 100.7075x; 2.5027x over previous
import functools

import jax
import jax.numpy as jnp
from jax import lax
from jax.experimental import pallas as pl
from jax.experimental.pallas import tpu as pltpu
from jax.experimental.pallas import tpu_sc as plsc

_NB = 256
_CHUNK = 2048
_NBINS = _NB * _NB
_NC, _NS = 2, 16


def _sc_histograms(e1, e2, e3, zeros_hbm, ones_hbm):
    E = e1.shape[1]
    NW = _NC * _NS
    CH = E // NW
    ROWS = CH // 128
    mesh = plsc.VectorSubcoreMesh(core_axis_name="c", subcore_axis_name="s")

    @functools.partial(
        pl.kernel, mesh=mesh,
        out_type=jax.ShapeDtypeStruct((_NC, 3 * _NBINS), jnp.float32),
        scratch_types=[
            pltpu.VMEM((CH,), jnp.int32),
            pltpu.VMEM((CH,), jnp.int32),
            pltpu.VMEM((3 * ROWS, 128), jnp.int32),
            pltpu.VMEM((128,), jnp.float32),
            pltpu.VMEM_SHARED((3 * _NBINS,), jnp.float32),
            pltpu.SemaphoreType.DMA,
        ],
    )
    def hist(e1_hbm, e2_hbm, e3_hbm, z_hbm, o_hbm, out_hbm,
             src_v, dst_v, idx_v, ones_v, shared, sem):
        c = lax.axis_index("c")
        s = lax.axis_index("s")
        base = (s * _NC + c) * CH

        @pl.when(s == 0)
        def _():
            pltpu.sync_copy(z_hbm, shared)

        pltpu.sync_copy(o_hbm, ones_v)
        for a, ehbm in enumerate((e1_hbm, e2_hbm, e3_hbm)):
            pltpu.sync_copy(ehbm.at[0, pl.ds(base, CH)], src_v)
            pltpu.sync_copy(ehbm.at[1, pl.ds(base, CH)], dst_v)

            def body(r, _, a=a):
                for u in range(8):
                    off = r * 128 + u * 16
                    sv = src_v[pl.ds(off, 16)]
                    dv = dst_v[pl.ds(off, 16)]
                    idx_v[a * ROWS + r, pl.ds(u * 16, 16)] = (
                        dv * _NB + sv + a * _NBINS)
                return 0

            lax.fori_loop(0, ROWS, body, 0)
        plsc.subcore_barrier()

        def sbody(j, _):
            pltpu.sync_copy(ones_v, shared.at[idx_v.at[j]], add=True)
            return 0

        lax.fori_loop(0, 3 * ROWS, sbody, 0)
        plsc.subcore_barrier()

        @pl.when(s == 0)
        def _():
            pltpu.sync_copy(shared, out_hbm.at[c])

    return hist(e1, e2, e3, zeros_hbm, ones_hbm)


def _hist_kernel(s1, d1, s2, d2, s3, d3, c1, c2, c3):
    step = pl.program_id(0)

    def onehot(idx_ref):
        idx = idx_ref[0]
        rows = jax.lax.broadcasted_iota(jnp.int32, (_NB, _CHUNK), 0)
        return jnp.where(idx == rows, 1.0, 0.0).astype(jnp.bfloat16)

    dn = (((1,), (1,)), ((), ()))
    for sref, dref, cref in ((s1, d1, c1), (s2, d2, c2), (s3, d3, c3)):
        oh_s = onehot(sref)
        oh_d = onehot(dref)
        prod = jax.lax.dot_general(oh_d, oh_s, dn,
                                   preferred_element_type=jnp.float32)

        @pl.when(step == 0)
        def _(cref=cref, prod=prod):
            cref[...] = prod

        @pl.when(step != 0)
        def _(cref=cref, prod=prod):
            cref[...] += prod


def _histograms(src1, dst1, src2, dst2, src3, dst3):
    e = src1.shape[0]
    steps = e // _CHUNK
    r = lambda a: a.reshape(steps, 1, _CHUNK)
    in_spec = pl.BlockSpec((1, 1, _CHUNK), lambda i: (i, 0, 0))
    out_spec = pl.BlockSpec((_NB, _NB), lambda i: (0, 0))
    return pl.pallas_call(
        _hist_kernel,
        grid=(steps,),
        in_specs=[in_spec] * 6,
        out_specs=[out_spec] * 3,
        out_shape=[jax.ShapeDtypeStruct((_NB, _NB), jnp.float32)] * 3,
    )(r(src1), r(dst1), r(src2), r(dst2), r(src3), r(dst3))


def _leaky(v):
    return jnp.where(v >= 0, v, 0.01 * v)


def _dense_kernel(c1, c2, c3, ne, w_h, b_h, w_l1, b_l1, w_gcn, b_gcn,
                  w_l2, b_l2, w_l3, b_l3, w_o, b_o, out, yt_scr):
    f32 = jnp.float32
    dn_t = (((0,), (0,)), ((), ()))
    ones_col = jnp.ones((_NB, 1), f32)

    C = c1[0] + c1[1]
    A = c2[0] + c2[1]
    Mc = c3[0] + c3[1]

    Bdeg = jnp.dot(C, ones_col, preferred_element_type=f32)
    Ddeg = jax.lax.dot_general(C, ones_col, dn_t,
                               preferred_element_type=f32)
    Binv = jnp.where(Bdeg > 0, 1.0 / Bdeg, 0.0)
    Dinv = jnp.where(Ddeg > 0, 1.0 / Ddeg, 0.0)

    x = jnp.dot(ne[...], w_h[...], preferred_element_type=f32)
    e = jnp.dot(C, x, preferred_element_type=f32) * Binv
    y = jax.lax.dot_general(C, e, dn_t, preferred_element_type=f32) * Dinv + b_h[...]
    y = _leaky(y)
    y = jax.lax.dot_general(y, y, dn_t, preferred_element_type=f32)
    y = _leaky(jnp.dot(y, w_l1[...], preferred_element_type=f32) + b_l1[...])

    ymean = jnp.dot(C, y, preferred_element_type=f32) / jnp.maximum(Bdeg, 1.0)
    xw = jnp.dot(ymean, w_gcn[...], preferred_element_type=f32)
    deg = jnp.dot(A, ones_col, preferred_element_type=f32) + 1.0
    dis = jnp.where(deg > 0, jax.lax.rsqrt(deg), 0.0)
    z = dis * xw
    yg = dis * (jnp.dot(A, z, preferred_element_type=f32) + z) + b_gcn[...]
    y = _leaky(jnp.dot(yg, w_l2[...], preferred_element_type=f32) + b_l2[...])

    big = jnp.float32(1e30)
    pen = jnp.where(Mc > 0, 0.0, big)
    yt_scr[...] = y

    def nstep(nb, acc):
        sel0 = jax.lax.broadcasted_iota(jnp.int32, (_NB, 8), 0)
        sel1 = jax.lax.broadcasted_iota(jnp.int32, (_NB, 8), 1)
        sel = jnp.where(sel0 == nb * 8 + sel1, 1.0, 0.0)
        pcols = jnp.dot(pen, sel, preferred_element_type=f32)
        yblk = yt_scr[pl.ds(nb * 8, 8), :]
        for u in range(8):
            cand = pcols[:, u:u + 1] + yblk[u:u + 1, :]
            acc = jnp.minimum(acc, cand)
        return acc

    acc = jax.lax.fori_loop(0, _NB // 8, nstep,
                            jnp.full((_NB, _NB), big, f32))
    ymin = jnp.where(acc > 1e29, 0.0, acc)
    ym = jnp.dot(ymin, w_l3[...], preferred_element_type=f32) + b_l3[...]
    out[...] = jnp.dot(ym, w_o[...], preferred_element_type=f32) + b_o[...]


def _dense(counts, ne, *weights):
    c1, c2, c3 = counts
    args = (c1, c2, c3, ne) + tuple(weights)
    return pl.pallas_call(
        _dense_kernel,
        out_shape=jax.ShapeDtypeStruct((_NB, 64), jnp.float32),
        scratch_shapes=[pltpu.VMEM((_NB, _NB), jnp.float32)],
    )(*args)


def kernel(edge_index, edge_edge_index, motif_edge_index, node_embeds,
           W_h, b_h, W_l1, b_l1, W_gcn, b_gcn, W_l2, b_l2, W_eo, b_eo,
           W_l3, b_l3, W_o, b_o):
    del W_eo, b_eo
    zeros_hbm = jnp.zeros((3 * _NBINS,), jnp.float32)
    ones_hbm = jnp.ones((128,), jnp.float32)
    hp = _sc_histograms(edge_index, edge_edge_index, motif_edge_index,
                        zeros_hbm, ones_hbm)
    hp = hp.reshape(_NC, 3, _NB, _NB)
    counts = (hp[:, 0], hp[:, 1], hp[:, 2])
    row = lambda b: b.reshape(1, -1)
    return _dense(counts, node_embeds,
                  W_h, row(b_h), W_l1, row(b_l1), W_gcn, row(b_gcn),
                  W_l2, row(b_l2), W_l3, row(b_l3), W_o, row(b_o))
